# Initial kernel scaffold; baseline (speedup 1.0000x reference)
import jax
import jax.numpy as jnp
from jax.experimental import pallas as pl


def kernel(domain_id, embedding):
    raise NotImplementedError("write your pallas kernel here")


# speedup vs baseline: 2.3440x; 2.3440x over previous
import functools

import jax
import jax.numpy as jnp
from jax import lax
from jax.experimental import pallas as pl
from jax.experimental.pallas import tpu as pltpu
from jax.experimental.pallas import tpu_sc as plsc

DIM = 128
NC = 2
NS = 16
NW = NC * NS
CHUNK = 128


@functools.lru_cache(maxsize=None)
def _make_kernel(B):
    b_per_w = B // NW
    n_chunks = b_per_w // CHUNK
    mesh = plsc.VectorSubcoreMesh(core_axis_name="c", subcore_axis_name="s")

    @functools.partial(
        pl.kernel,
        mesh=mesh,
        out_type=jax.ShapeDtypeStruct((NW, n_chunks, CHUNK, DIM), jnp.float32),
        scratch_types=[
            pltpu.VMEM((n_chunks, CHUNK), jnp.int32),
            pltpu.VMEM((n_chunks, CHUNK, DIM), jnp.float32),
            pltpu.SemaphoreType.DMA,
            pltpu.SemaphoreType.DMA,
        ],
    )
    def gather_kernel(table_hbm, idx_hbm, out_hbm, idx_v, rows_v, gsem, ssem):
        wid = lax.axis_index("s") * NC + lax.axis_index("c")
        pltpu.sync_copy(idx_hbm.at[wid], idx_v)
        gathers = [
            pltpu.async_copy(table_hbm.at[idx_v.at[j]], rows_v.at[j], gsem)
            for j in range(n_chunks)
        ]
        stores = []
        for j in range(n_chunks):
            gathers[j].wait()
            stores.append(pltpu.async_copy(rows_v.at[j], out_hbm.at[wid, j], ssem))
        for s in stores:
            s.wait()

    return gather_kernel


def kernel(domain_id, embedding):
    B = domain_id.shape[0]
    idx = domain_id.astype(jnp.int32).reshape(NW, B // NW // CHUNK, CHUNK)
    out = _make_kernel(B)(embedding, idx)
    return out.reshape(B, DIM)


# speedup vs baseline: 2.8483x; 1.2152x over previous
import functools

import jax
import jax.numpy as jnp
from jax import lax
from jax.experimental import pallas as pl
from jax.experimental.pallas import tpu as pltpu
from jax.experimental.pallas import tpu_sc as plsc

DIM = 128
NC = 2
NS = 16
NW = NC * NS
CHUNK = 128
VOCAB = 1000
ROWS_PER_TILE = 64


@functools.lru_cache(maxsize=None)
def _make_kernel(B):
    b_per_w = B // NW
    n_chunks = b_per_w // CHUNK
    mesh = plsc.VectorSubcoreMesh(core_axis_name="c", subcore_axis_name="s")

    @functools.partial(
        pl.kernel,
        mesh=mesh,
        out_type=jax.ShapeDtypeStruct((NW, n_chunks, CHUNK, DIM), jnp.float32),
        scratch_types=[
            pltpu.VMEM_SHARED((VOCAB, DIM), jnp.float32),
            pltpu.VMEM((n_chunks, CHUNK), jnp.int32),
            pltpu.VMEM((n_chunks, CHUNK, DIM), jnp.float32),
            pltpu.SemaphoreType.DMA,
            pltpu.SemaphoreType.DMA,
            pltpu.SemaphoreType.DMA,
        ],
    )
    def gather_kernel(table_hbm, idx_hbm, out_hbm, table_sh, idx_v, rows_v,
                      isem, gsem, ssem):
        cid = lax.axis_index("c")
        sid = lax.axis_index("s")
        wid = sid * NC + cid
        idx_cp = pltpu.async_copy(idx_hbm.at[wid], idx_v, isem)
        row0 = jnp.minimum(sid * ROWS_PER_TILE, VOCAB - ROWS_PER_TILE)
        row0 = pl.multiple_of(row0, 8)
        pltpu.sync_copy(table_hbm.at[pl.ds(row0, ROWS_PER_TILE)],
                        table_sh.at[pl.ds(row0, ROWS_PER_TILE)])
        idx_cp.wait()
        plsc.subcore_barrier()
        gathers = [
            pltpu.async_copy(table_sh.at[idx_v.at[j]], rows_v.at[j], gsem)
            for j in range(n_chunks)
        ]
        stores = []
        for j in range(n_chunks):
            gathers[j].wait()
            stores.append(pltpu.async_copy(rows_v.at[j], out_hbm.at[wid, j], ssem))
        for s in stores:
            s.wait()

    return gather_kernel


def kernel(domain_id, embedding):
    B = domain_id.shape[0]
    idx = domain_id.astype(jnp.int32).reshape(NW, B // NW // CHUNK, CHUNK)
    out = _make_kernel(B)(embedding, idx)
    return out.reshape(B, DIM)
